# baseline (speedup 1.0000x reference)
import numpy as np

import jax
import jax.numpy as jnp
from jax.experimental import pallas as pl
from jax.experimental.shard_map import shard_map
from jax.sharding import Mesh, PartitionSpec as P

N, F, H1, H2, C = 10000, 256, 256, 256, 64


def _small_matmul_kernel(x_ref, w_ref, o_ref):
    a = x_ref[...].astype(jnp.bfloat16)
    b = w_ref[...].astype(jnp.bfloat16)
    o_ref[...] = jnp.dot(a, b, preferred_element_type=jnp.float32).astype(
        jnp.bfloat16
    )


def _small_matmul(x, w, bm=1000):
    m, k = x.shape
    _, n = w.shape
    return pl.pallas_call(
        _small_matmul_kernel,
        grid=(m // bm,),
        in_specs=[
            pl.BlockSpec((bm, k), lambda i: (i, 0)),
            pl.BlockSpec((k, n), lambda i: (0, 0)),
        ],
        out_specs=pl.BlockSpec((bm, n), lambda i: (i, 0)),
        out_shape=jax.ShapeDtypeStruct((m, n), jnp.bfloat16),
    )(x, w)


def _layer_mid_kernel(adj_ref, sup_ref, b_ref, w_next_ref, o_ref):
    a = adj_ref[...].astype(jnp.bfloat16)
    h = jnp.dot(a, sup_ref[...], preferred_element_type=jnp.float32)
    h = jnp.maximum(h + b_ref[...], 0.0)
    o_ref[...] = jnp.dot(
        h.astype(jnp.bfloat16), w_next_ref[...], preferred_element_type=jnp.float32
    ).astype(jnp.bfloat16)


def _layer_mid(adj, support, b, w_next, bm):
    m = adj.shape[0]
    n_out = w_next.shape[1]
    return pl.pallas_call(
        _layer_mid_kernel,
        grid=(m // bm,),
        in_specs=[
            pl.BlockSpec((bm, N), lambda i: (i, 0)),
            pl.BlockSpec((N, H1), lambda i: (0, 0)),
            pl.BlockSpec((1, H1), lambda i: (0, 0)),
            pl.BlockSpec((H1, n_out), lambda i: (0, 0)),
        ],
        out_specs=pl.BlockSpec((bm, n_out), lambda i: (i, 0)),
        out_shape=jax.ShapeDtypeStruct((m, n_out), jnp.bfloat16),
    )(adj, support, b.reshape(1, -1), w_next)


def _layer_final_kernel(adj_ref, sup_ref, b_ref, wf_ref, bf_ref, o_ref):
    a = adj_ref[...].astype(jnp.bfloat16)
    h = jnp.dot(a, sup_ref[...], preferred_element_type=jnp.float32)
    h = jnp.maximum(h + b_ref[...], 0.0)
    logits = (
        jnp.dot(h.astype(jnp.bfloat16), wf_ref[...], preferred_element_type=jnp.float32)
        + bf_ref[...]
    )
    m = jnp.max(logits, axis=1, keepdims=True)
    s = logits - m
    lse = jnp.log(jnp.sum(jnp.exp(s), axis=1, keepdims=True))
    o_ref[...] = s - lse


def _layer_final(adj, support, b, wf, bfin, bm):
    m = adj.shape[0]
    return pl.pallas_call(
        _layer_final_kernel,
        grid=(m // bm,),
        in_specs=[
            pl.BlockSpec((bm, N), lambda i: (i, 0)),
            pl.BlockSpec((N, H2), lambda i: (0, 0)),
            pl.BlockSpec((1, H2), lambda i: (0, 0)),
            pl.BlockSpec((H2, C), lambda i: (0, 0)),
            pl.BlockSpec((1, C), lambda i: (0, 0)),
        ],
        out_specs=pl.BlockSpec((bm, C), lambda i: (i, 0)),
        out_shape=jax.ShapeDtypeStruct((m, C), jnp.float32),
    )(adj, support, b.reshape(1, -1), wf, bfin.reshape(1, -1))


def _forward(x, adj_local, W1, b1, W2, b2, Wf, bf, bm, gather):
    support1 = _small_matmul(x, W1)
    support2 = _layer_mid(adj_local, support1, b1, W2, bm)
    support2 = gather(support2)
    return _layer_final(adj_local, support2, b2, Wf, bf, bm)


def kernel(x, adj, W1, b1, W2, b2, Wf, bf):
    devs = jax.devices()
    nd = 2 if len(devs) >= 2 and N % (2 * 8) == 0 else 1
    if nd == 1:
        return _forward(
            x, adj, W1, b1, W2, b2, Wf, bf, 400, lambda s: s
        )
    mesh = Mesh(np.array(devs[:nd]), ("d",))

    def impl(x, adj_local, W1, b1, W2, b2, Wf, bf):
        return _forward(
            x, adj_local, W1, b1, W2, b2, Wf, bf, 200,
            lambda s: jax.lax.all_gather(s, "d", axis=0, tiled=True),
        )

    f = shard_map(
        impl,
        mesh=mesh,
        in_specs=(P(), P("d", None), P(), P(), P(), P(), P(), P()),
        out_specs=P("d", None),
        check_rep=False,
    )
    return f(x, adj, W1, b1, W2, b2, Wf, bf)


# speedup vs baseline: 3.0354x; 3.0354x over previous
import jax
import jax.numpy as jnp
from jax.experimental import pallas as pl

N, F, H1, H2, C = 10000, 256, 256, 256, 64

BM = 200


def _small_matmul_kernel(x_ref, w_ref, o_ref):
    a = x_ref[...].astype(jnp.bfloat16)
    b = w_ref[...].astype(jnp.bfloat16)
    o_ref[...] = jnp.dot(a, b, preferred_element_type=jnp.float32).astype(
        jnp.bfloat16
    )


def _small_matmul(x, w, bm=1000):
    m, k = x.shape
    _, n = w.shape
    return pl.pallas_call(
        _small_matmul_kernel,
        grid=(m // bm,),
        in_specs=[
            pl.BlockSpec((bm, k), lambda i: (i, 0)),
            pl.BlockSpec((k, n), lambda i: (0, 0)),
        ],
        out_specs=pl.BlockSpec((bm, n), lambda i: (i, 0)),
        out_shape=jax.ShapeDtypeStruct((m, n), jnp.bfloat16),
    )(x, w)


def _layer_mid_kernel(adj_ref, sup_ref, b_ref, w_next_ref, o_ref):
    a = adj_ref[...].astype(jnp.bfloat16)
    h = jnp.dot(a, sup_ref[...], preferred_element_type=jnp.float32)
    h = jnp.maximum(h + b_ref[...], 0.0)
    o_ref[...] = jnp.dot(
        h.astype(jnp.bfloat16), w_next_ref[...], preferred_element_type=jnp.float32
    ).astype(jnp.bfloat16)


def _layer_mid(adj, support, b, w_next):
    n_out = w_next.shape[1]
    return pl.pallas_call(
        _layer_mid_kernel,
        grid=(N // BM,),
        in_specs=[
            pl.BlockSpec((BM, N), lambda i: (i, 0)),
            pl.BlockSpec((N, H1), lambda i: (0, 0)),
            pl.BlockSpec((1, H1), lambda i: (0, 0)),
            pl.BlockSpec((H1, n_out), lambda i: (0, 0)),
        ],
        out_specs=pl.BlockSpec((BM, n_out), lambda i: (i, 0)),
        out_shape=jax.ShapeDtypeStruct((N, n_out), jnp.bfloat16),
    )(adj, support, b.reshape(1, -1), w_next)


def _layer_final_kernel(adj_ref, sup_ref, b_ref, wf_ref, bf_ref, o_ref):
    a = adj_ref[...].astype(jnp.bfloat16)
    h = jnp.dot(a, sup_ref[...], preferred_element_type=jnp.float32)
    h = jnp.maximum(h + b_ref[...], 0.0)
    logits = (
        jnp.dot(h.astype(jnp.bfloat16), wf_ref[...], preferred_element_type=jnp.float32)
        + bf_ref[...]
    )
    m = jnp.max(logits, axis=1, keepdims=True)
    s = logits - m
    lse = jnp.log(jnp.sum(jnp.exp(s), axis=1, keepdims=True))
    o_ref[...] = s - lse


def _layer_final(adj, support, b, wf, bfin):
    return pl.pallas_call(
        _layer_final_kernel,
        grid=(N // BM,),
        in_specs=[
            pl.BlockSpec((BM, N), lambda i: (i, 0)),
            pl.BlockSpec((N, H2), lambda i: (0, 0)),
            pl.BlockSpec((1, H2), lambda i: (0, 0)),
            pl.BlockSpec((H2, C), lambda i: (0, 0)),
            pl.BlockSpec((1, C), lambda i: (0, 0)),
        ],
        out_specs=pl.BlockSpec((BM, C), lambda i: (i, 0)),
        out_shape=jax.ShapeDtypeStruct((N, C), jnp.float32),
    )(adj, support, b.reshape(1, -1), wf, bfin.reshape(1, -1))


def kernel(x, adj, W1, b1, W2, b2, Wf, bf):
    support1 = _small_matmul(x, W1)
    support2 = _layer_mid(adj, support1, b1, W2)
    return _layer_final(adj, support2, b2, Wf, bf)


# speedup vs baseline: 3.2237x; 1.0620x over previous
import jax
import jax.numpy as jnp
from jax.experimental import pallas as pl
from jax.experimental.pallas import tpu as pltpu

N, F, H1, H2, C = 10000, 256, 256, 256, 64

BM = 400
NBLK = N // BM


def _small_matmul_kernel(x_ref, w_ref, o_ref):
    a = x_ref[...].astype(jnp.bfloat16)
    b = w_ref[...].astype(jnp.bfloat16)
    o_ref[...] = jnp.dot(a, b, preferred_element_type=jnp.float32).astype(
        jnp.bfloat16
    )


def _small_matmul(x, w, bm=1000):
    m, k = x.shape
    _, n = w.shape
    return pl.pallas_call(
        _small_matmul_kernel,
        grid=(m // bm,),
        in_specs=[
            pl.BlockSpec((bm, k), lambda i: (i, 0)),
            pl.BlockSpec((k, n), lambda i: (0, 0)),
        ],
        out_specs=pl.BlockSpec((bm, n), lambda i: (i, 0)),
        out_shape=jax.ShapeDtypeStruct((m, n), jnp.bfloat16),
    )(x, w)


def _merged_kernel(
    adj_ref, sup1_ref, b1_ref, w2_ref, b2_ref, wf_ref, bf_ref, o_ref, s2_ref
):
    i = pl.program_id(0)
    a = adj_ref[...].astype(jnp.bfloat16)

    @pl.when(i < NBLK)
    def _phase_a():
        h = jnp.dot(a, sup1_ref[...], preferred_element_type=jnp.float32)
        h = jnp.maximum(h + b1_ref[...], 0.0)
        s2 = jnp.dot(
            h.astype(jnp.bfloat16), w2_ref[...], preferred_element_type=jnp.float32
        )
        s2_ref[pl.ds(i * BM, BM), :] = s2.astype(jnp.bfloat16)
        o_ref[...] = jnp.zeros_like(o_ref)

    @pl.when(i >= NBLK)
    def _phase_b():
        h = jnp.dot(a, s2_ref[...], preferred_element_type=jnp.float32)
        h = jnp.maximum(h + b2_ref[...], 0.0)
        logits = (
            jnp.dot(
                h.astype(jnp.bfloat16),
                wf_ref[...],
                preferred_element_type=jnp.float32,
            )
            + bf_ref[...]
        )
        m = jnp.max(logits, axis=1, keepdims=True)
        s = logits - m
        lse = jnp.log(jnp.sum(jnp.exp(s), axis=1, keepdims=True))
        o_ref[...] = s - lse


def kernel(x, adj, W1, b1, W2, b2, Wf, bf):
    support1 = _small_matmul(x, W1)
    return pl.pallas_call(
        _merged_kernel,
        grid=(2 * NBLK,),
        in_specs=[
            pl.BlockSpec((BM, N), lambda i: (i % NBLK, 0)),
            pl.BlockSpec((N, H1), lambda i: (0, 0)),
            pl.BlockSpec((1, H1), lambda i: (0, 0)),
            pl.BlockSpec((H1, H2), lambda i: (0, 0)),
            pl.BlockSpec((1, H2), lambda i: (0, 0)),
            pl.BlockSpec((H2, C), lambda i: (0, 0)),
            pl.BlockSpec((1, C), lambda i: (0, 0)),
        ],
        out_specs=pl.BlockSpec((BM, C), lambda i: (i % NBLK, 0)),
        out_shape=jax.ShapeDtypeStruct((N, C), jnp.float32),
        scratch_shapes=[pltpu.VMEM((N, H2), jnp.bfloat16)],
    )(
        adj,
        support1,
        b1.reshape(1, -1),
        W2,
        b2.reshape(1, -1),
        Wf,
        bf.reshape(1, -1),
    )
